# baseline (speedup 1.0000x reference)
import jax
import jax.numpy as jnp
from jax import lax
from jax.experimental import pallas as pl
from jax.experimental.pallas import tpu as pltpu
from jax.experimental.pallas import tpu_sc as plsc

N_NODES = 10000
N_EDGES = 320000
D = 128

NC = 2
NS = 16
NW = NC * NS
E_PER_TILE = N_EDGES // NW
CHUNK = 128
N_CHUNKS = 80
E_PAD = N_CHUNKS * CHUNK
NJ = N_CHUNKS // 2
N_PAD = 10240
ROWS_PER_TILE = N_PAD // NS


def _sc_body(feat_hbm, idx_hbm, zeros_hbm, out_hbm,
             idxp, rows0, rows1, acc, sg0, sg1):
    cid = lax.axis_index("c")
    sid = lax.axis_index("s")
    wid = cid * NS + sid
    row_base = sid * ROWS_PER_TILE

    pltpu.sync_copy(zeros_hbm, acc.at[pl.ds(row_base, ROWS_PER_TILE)])
    plsc.subcore_barrier()

    def step(j, carry):
        pltpu.sync_copy(idx_hbm.at[wid, j], idxp)
        d0 = pltpu.async_copy(feat_hbm.at[idxp.at[0, 0]], rows0, sg0)
        d1 = pltpu.async_copy(feat_hbm.at[idxp.at[1, 0]], rows1, sg1)
        d0.wait()
        pltpu.sync_copy(rows0, acc.at[idxp.at[0, 1]], add=True)
        d1.wait()
        pltpu.sync_copy(rows1, acc.at[idxp.at[1, 1]], add=True)
        return carry

    lax.fori_loop(0, NJ, step, 0)

    plsc.subcore_barrier()
    pltpu.sync_copy(acc.at[pl.ds(row_base, ROWS_PER_TILE)],
                    out_hbm.at[cid, pl.ds(row_base, ROWS_PER_TILE)])


def _sc_aggregate(features, idx):
    mesh = plsc.VectorSubcoreMesh(core_axis_name="c", subcore_axis_name="s")
    zeros = jnp.zeros((ROWS_PER_TILE, D), jnp.float32)
    return pl.kernel(
        _sc_body,
        out_type=jax.ShapeDtypeStruct((NC, N_PAD, D), jnp.float32),
        mesh=mesh,
        scratch_types=[
            pltpu.VMEM((2, 2, CHUNK), jnp.int32),
            pltpu.VMEM((CHUNK, D), jnp.float32),
            pltpu.VMEM((CHUNK, D), jnp.float32),
            pltpu.VMEM_SHARED((N_PAD, D), jnp.float32),
            pltpu.SemaphoreType.DMA,
            pltpu.SemaphoreType.DMA,
        ],
    )(features, idx, zeros)


ROW_BLK = 1000


def _tc_body(p_ref, w_ref, b_ref, o_ref):
    agg = p_ref[0] + p_ref[1]
    o_ref[...] = (
        jnp.dot(agg, w_ref[...], preferred_element_type=jnp.float32)
        + b_ref[...]
    )


def _tc_linear(partials, W, b):
    return pl.pallas_call(
        _tc_body,
        grid=(N_NODES // ROW_BLK,),
        in_specs=[
            pl.BlockSpec((NC, ROW_BLK, D), lambda i: (0, i, 0)),
            pl.BlockSpec((D, D), lambda i: (0, 0)),
            pl.BlockSpec((1, D), lambda i: (0, 0)),
        ],
        out_specs=pl.BlockSpec((ROW_BLK, D), lambda i: (i, 0)),
        out_shape=jax.ShapeDtypeStruct((N_NODES, D), jnp.float32),
    )(partials, W, b.reshape(1, D))


def kernel(features, edge_index, W, b):
    src = edge_index[0].astype(jnp.int32).reshape(NW, E_PER_TILE)
    dst = edge_index[1].astype(jnp.int32).reshape(NW, E_PER_TILE)
    pad = E_PAD - E_PER_TILE
    src3 = jnp.pad(src, ((0, 0), (0, pad))).reshape(NW, N_CHUNKS, CHUNK)
    dst3 = jnp.pad(dst, ((0, 0), (0, pad)),
                   constant_values=N_NODES).reshape(NW, N_CHUNKS, CHUNK)
    idx = jnp.stack([src3, dst3], axis=2).reshape(NW, NJ, 2, 2, CHUNK)
    partials = _sc_aggregate(features, idx)
    return _tc_linear(partials, W, b)


# speedup vs baseline: 2.2805x; 2.2805x over previous
import jax
import jax.numpy as jnp
from jax import lax
from jax.experimental import pallas as pl
from jax.experimental.pallas import tpu as pltpu
from jax.experimental.pallas import tpu_sc as plsc

N_NODES = 10000
N_EDGES = 320000
D = 128

NC = 2
NS = 16
NW = NC * NS
E_PER_TILE = N_EDGES // NW
CHUNK = 128
N_CHUNKS = 80
E_PAD = N_CHUNKS * CHUNK
NJ = N_CHUNKS // 2
N_PAD = 10240
ROWS_PER_TILE = N_PAD // NS


def _sc_body(feat_hbm, idx_hbm, zeros_hbm, out_hbm,
             idxp, rows0, rows1, acc, sg0, sg1):
    cid = lax.axis_index("c")
    sid = lax.axis_index("s")
    wid = cid * NS + sid
    row_base = sid * ROWS_PER_TILE

    pltpu.sync_copy(zeros_hbm, acc.at[pl.ds(row_base, ROWS_PER_TILE)])
    plsc.subcore_barrier()

    def step(j, carry):
        pltpu.sync_copy(idx_hbm.at[wid, j], idxp)
        d0 = pltpu.async_copy(feat_hbm.at[idxp.at[0, 0]], rows0, sg0)
        d1 = pltpu.async_copy(feat_hbm.at[idxp.at[1, 0]], rows1, sg1)
        d0.wait()
        pltpu.sync_copy(rows0, acc.at[idxp.at[0, 1]], add=True)
        d1.wait()
        pltpu.sync_copy(rows1, acc.at[idxp.at[1, 1]], add=True)
        return carry

    lax.fori_loop(0, NJ, step, 0)

    plsc.subcore_barrier()
    pltpu.sync_copy(acc.at[pl.ds(row_base, ROWS_PER_TILE)],
                    out_hbm.at[cid, pl.ds(row_base, ROWS_PER_TILE)])


def _sc_aggregate(features, idx):
    mesh = plsc.VectorSubcoreMesh(core_axis_name="c", subcore_axis_name="s")
    zeros = jnp.zeros((ROWS_PER_TILE, D), jnp.float32)
    return pl.kernel(
        _sc_body,
        out_type=jax.ShapeDtypeStruct((NC, N_PAD, D), jnp.float32),
        mesh=mesh,
        scratch_types=[
            pltpu.VMEM((2, 2, CHUNK), jnp.int32),
            pltpu.VMEM((CHUNK, D), jnp.float32),
            pltpu.VMEM((CHUNK, D), jnp.float32),
            pltpu.VMEM_SHARED((N_PAD, D), jnp.float32),
            pltpu.SemaphoreType.DMA,
            pltpu.SemaphoreType.DMA,
        ],
    )(features, idx, zeros)


ROW_BLK = 1000


def _tc_body(p_ref, w_ref, b_ref, o_ref):
    agg = p_ref[0] + p_ref[1]
    o_ref[...] = (
        jnp.dot(agg, w_ref[...], preferred_element_type=jnp.float32)
        + b_ref[...]
    )


def _tc_linear(partials, W, b):
    return pl.pallas_call(
        _tc_body,
        grid=(N_NODES // ROW_BLK,),
        in_specs=[
            pl.BlockSpec((NC, ROW_BLK, D), lambda i: (0, i, 0)),
            pl.BlockSpec((D, D), lambda i: (0, 0)),
            pl.BlockSpec((1, D), lambda i: (0, 0)),
        ],
        out_specs=pl.BlockSpec((ROW_BLK, D), lambda i: (i, 0)),
        out_shape=jax.ShapeDtypeStruct((N_NODES, D), jnp.float32),
    )(partials, W, b.reshape(1, D))


def kernel(features, edge_index, W, b):
    src = edge_index[0].astype(jnp.int32).reshape(NW, E_PER_TILE)
    dst = edge_index[1].astype(jnp.int32).reshape(NW, E_PER_TILE)
    pad = E_PAD - E_PER_TILE
    pad_src = (jnp.arange(pad, dtype=jnp.int32) * 41) % N_NODES
    pad_dst = N_NODES + (jnp.arange(pad, dtype=jnp.int32) % (N_PAD - N_NODES))
    src3 = jnp.concatenate(
        [src, jnp.broadcast_to(pad_src[None], (NW, pad))], axis=1
    ).reshape(NW, N_CHUNKS, CHUNK)
    dst3 = jnp.concatenate(
        [dst, jnp.broadcast_to(pad_dst[None], (NW, pad))], axis=1
    ).reshape(NW, N_CHUNKS, CHUNK)
    idx = jnp.stack([src3, dst3], axis=2).reshape(NW, NJ, 2, 2, CHUNK)
    partials = _sc_aggregate(features, idx)
    return _tc_linear(partials, W, b)
